# Initial kernel scaffold; baseline (speedup 1.0000x reference)
import jax
import jax.numpy as jnp
from jax.experimental import pallas as pl


def kernel(x, h, c, edge_index, W_ih, W_hh, b, W_role, b_role):
    raise NotImplementedError("write your pallas kernel here")


# speedup vs baseline: 6.0350x; 6.0350x over previous
import functools

import jax
import jax.numpy as jnp
from jax import lax
from jax.experimental import pallas as pl
from jax.experimental.pallas import tpu as pltpu
from jax.experimental.pallas import tpu_sc as plsc

N = 10000
E = 320000
D = 128
H = 128

NC = 2
NS = 16
NW = NC * NS
EPW = E // NW
EB = 80
NB = EPW // EB
RPS = 624
ZR = 48
NTAIL = N - NS * RPS


def _sc_gather_scatter(x, src, dst):
    mesh = plsc.VectorSubcoreMesh(core_axis_name="c", subcore_axis_name="s")

    @functools.partial(
        pl.kernel,
        out_type=(
            jax.ShapeDtypeStruct((NC, N, D), jnp.float32),
            jax.ShapeDtypeStruct((NC, NS, N), jnp.float32),
        ),
        mesh=mesh,
        compiler_params=pltpu.CompilerParams(needs_layout_passes=False),
        scratch_types=[
            pltpu.VMEM((EB,), jnp.int32),
            pltpu.VMEM((EB,), jnp.int32),
            pltpu.VMEM((EB, D), jnp.float32),
            pltpu.VMEM((ZR, D), jnp.float32),
            pltpu.VMEM((N,), jnp.float32),
            pltpu.VMEM_SHARED((N, D), jnp.float32),
            pltpu.SemaphoreType.DMA,
        ],
    )
    def body(x_hbm, src_hbm, dst_hbm, out_hbm, deg_hbm,
             sidx, didx, rows, zbuf, hist, acc, sem):
        c = lax.axis_index("c")
        s = lax.axis_index("s")
        w = c * NS + s

        z16 = jnp.zeros((16,), jnp.float32)
        for r in range(ZR):
            for k in range(D // 16):
                zbuf[r, pl.ds(k * 16, 16)] = z16

        def zstep(j, carry):
            pltpu.sync_copy(zbuf, acc.at[pl.ds(s * RPS + j * ZR, ZR)])
            return carry

        lax.fori_loop(0, RPS // ZR, zstep, 0)

        @pl.when(s == 0)
        def _():
            pltpu.sync_copy(zbuf.at[pl.ds(0, NTAIL)], acc.at[pl.ds(NS * RPS, NTAIL)])

        def hzstep(j, carry):
            hist[pl.ds(j * 16, 16)] = z16
            return carry

        lax.fori_loop(0, N // 16, hzstep, 0)
        plsc.subcore_barrier()

        base = w * EPW
        ones16 = jnp.ones((16,), jnp.float32)

        def estep(i, carry):
            off = base + i * EB
            pltpu.sync_copy(src_hbm.at[pl.ds(off, EB)], sidx)
            pltpu.sync_copy(dst_hbm.at[pl.ds(off, EB)], didx)
            pltpu.async_copy(x_hbm.at[sidx], rows, sem).wait()
            pltpu.sync_copy(rows, acc.at[didx], add=True)
            for k in range(EB // 16):
                idx16 = didx[pl.ds(k * 16, 16)]
                plsc.addupdate_scatter(hist, [idx16], ones16)
            return carry

        lax.fori_loop(0, NB, estep, 0)
        plsc.subcore_barrier()

        pltpu.sync_copy(hist, deg_hbm.at[c, s])

        @pl.when(s == 0)
        def _():
            pltpu.sync_copy(acc, out_hbm.at[c])

    return body(x, src, dst)


BN = 1000


def _tc_body(x_ref, h_ref, c_ref, a0_ref, a1_ref, deg_ref, w1_ref, w2_ref,
             whh_ref, b_ref, wr_ref, br_ref, nh_ref, nc_ref, rp_ref):
    deg = jnp.maximum(deg_ref[...], 1.0)
    agg = (a0_ref[...] + a1_ref[...]) / deg

    xb = x_ref[...]
    hb = h_ref[...]
    gates = (
        jnp.dot(xb, w1_ref[...], preferred_element_type=jnp.float32)
        + jnp.dot(agg, w2_ref[...], preferred_element_type=jnp.float32)
        + jnp.dot(hb, whh_ref[...], preferred_element_type=jnp.float32)
        + b_ref[...]
    )
    i_g = 1.0 / (1.0 + jnp.exp(-gates[:, 0 * H:1 * H]))
    f_g = 1.0 / (1.0 + jnp.exp(-gates[:, 1 * H:2 * H]))
    g_g = jnp.tanh(gates[:, 2 * H:3 * H])
    o_g = 1.0 / (1.0 + jnp.exp(-gates[:, 3 * H:4 * H]))
    new_c = f_g * c_ref[...] + i_g * g_g
    new_h = o_g * jnp.tanh(new_c)
    nc_ref[...] = new_c
    nh_ref[...] = new_h

    logits = jnp.dot(new_h, wr_ref[...], preferred_element_type=jnp.float32)
    logits = logits + br_ref[...]
    m = jnp.max(logits, axis=1, keepdims=True)
    e = jnp.exp(logits - m)
    probs = e / jnp.sum(e, axis=1, keepdims=True)
    rp_ref[...] = probs[:, :3]


def _tc_dense(x, h, c, a0, a1, deg, W1, W2, W_hh, b, Wr_pad, br_pad):
    grid = (N // BN,)
    blk = lambda rows, cols: pl.BlockSpec((rows, cols), lambda i: (i, 0))
    full = lambda rows, cols: pl.BlockSpec((rows, cols), lambda i: (0, 0))
    return pl.pallas_call(
        _tc_body,
        grid=grid,
        in_specs=[
            blk(BN, D),
            blk(BN, H),
            blk(BN, H),
            blk(BN, D),
            blk(BN, D),
            blk(BN, 1),
            full(D, 4 * H),
            full(D, 4 * H),
            full(H, 4 * H),
            pl.BlockSpec((4 * H,), lambda i: (0,)),
            full(H, 128),
            pl.BlockSpec((128,), lambda i: (0,)),
        ],
        out_specs=[
            blk(BN, H),
            blk(BN, H),
            blk(BN, 3),
        ],
        out_shape=[
            jax.ShapeDtypeStruct((N, H), jnp.float32),
            jax.ShapeDtypeStruct((N, H), jnp.float32),
            jax.ShapeDtypeStruct((N, 3), jnp.float32),
        ],
    )(x, h, c, a0, a1, deg, W1, W2, W_hh, b, Wr_pad, br_pad)


def kernel(x, h, c, edge_index, W_ih, W_hh, b, W_role, b_role):
    src = edge_index[0]
    dst = edge_index[1]
    part, deg_tiles = _sc_gather_scatter(x, src, dst)
    deg = jnp.sum(deg_tiles, axis=(0, 1)).reshape(N, 1)

    W1 = W_ih[:D]
    W2 = W_ih[D:]
    Wr_pad = jnp.zeros((H, 128), jnp.float32).at[:, :3].set(W_role)
    br_pad = jnp.full((128,), -1e30, jnp.float32).at[:3].set(b_role)

    new_h, new_c, role_probs = _tc_dense(
        x, h, c, part[0], part[1], deg, W1, W2, W_hh, b, Wr_pad, br_pad
    )
    return new_h, new_c, role_probs
